# baseline (speedup 1.0000x reference)
import jax
import jax.numpy as jnp
from jax import lax
from jax.experimental import pallas as pl
from jax.experimental.pallas import tpu as pltpu
from jax.experimental.pallas import tpu_sc as plsc

NC = 2
NS = 16
L = 16
NW = NC * NS
B = 16384
D = 32
BPW = B // NW
CHUNK = 128
NCH = BPW // CHUNK
GROUPS = BPW // L


def _sc_kernel(users_hbm, pos_hbm, neg_hbm, uet_hbm, iet_hbm, ub_hbm, ib_hbm,
               outp_hbm, outn_hbm,
               uidx, pidx, nidx, ue_fac, pe_fac, ne_fac,
               ub_v, pb_v, nb_v, outp_v, outn_v, sem):
    wid = lax.axis_index("core") * NS + lax.axis_index("subcore")

    pltpu.sync_copy(users_hbm.at[wid], uidx)
    pltpu.sync_copy(pos_hbm.at[wid], pidx)
    pltpu.sync_copy(neg_hbm.at[wid], nidx)

    copies = []
    for j in range(NCH):
        sl = pl.ds(j * CHUNK, CHUNK)
        copies.append(pltpu.async_copy(ub_hbm.at[uidx.at[j]], ub_v.at[sl], sem))
        copies.append(pltpu.async_copy(ib_hbm.at[pidx.at[j]], pb_v.at[sl], sem))
        copies.append(pltpu.async_copy(ib_hbm.at[nidx.at[j]], nb_v.at[sl], sem))
        for d in range(D):
            copies.append(pltpu.async_copy(uet_hbm.at[d].at[uidx.at[j]],
                                           ue_fac.at[d].at[sl], sem))
            copies.append(pltpu.async_copy(iet_hbm.at[d].at[pidx.at[j]],
                                           pe_fac.at[d].at[sl], sem))
            copies.append(pltpu.async_copy(iet_hbm.at[d].at[nidx.at[j]],
                                           ne_fac.at[d].at[sl], sem))
    for c in copies:
        c.wait()

    @pl.loop(0, GROUPS)
    def _group(g):
        sl = pl.ds(g * L, L)
        accp = ub_v[sl] + pb_v[sl]
        accn = ub_v[sl] + nb_v[sl]
        for d in range(D):
            u = ue_fac[d, sl]
            accp = accp + u * pe_fac[d, sl]
            accn = accn + u * ne_fac[d, sl]
        outp_v[sl] = accp
        outn_v[sl] = accn

    pltpu.sync_copy(outp_v, outp_hbm.at[pl.ds(wid * BPW, BPW)])
    pltpu.sync_copy(outn_v, outn_hbm.at[pl.ds(wid * BPW, BPW)])


def kernel(users, pos_items, neg_items, user_embeddings, item_embeddings,
           user_biases, item_biases):
    users3 = users.astype(jnp.int32).reshape(NW, NCH, CHUNK)
    pos3 = pos_items.astype(jnp.int32).reshape(NW, NCH, CHUNK)
    neg3 = neg_items.astype(jnp.int32).reshape(NW, NCH, CHUNK)
    uet = user_embeddings.T
    iet = item_embeddings.T
    ub1 = user_biases.reshape(-1)
    ib1 = item_biases.reshape(-1)

    mesh = plsc.VectorSubcoreMesh(core_axis_name="core",
                                  subcore_axis_name="subcore",
                                  num_cores=NC, num_subcores=NS)
    f = pl.kernel(
        _sc_kernel,
        compiler_params=pltpu.CompilerParams(needs_layout_passes=False,
                                             use_tc_tiling_on_sc=False),
        out_type=(jax.ShapeDtypeStruct((B,), jnp.float32),
                  jax.ShapeDtypeStruct((B,), jnp.float32)),
        mesh=mesh,
        scratch_types=[
            pltpu.VMEM((NCH, CHUNK), jnp.int32),
            pltpu.VMEM((NCH, CHUNK), jnp.int32),
            pltpu.VMEM((NCH, CHUNK), jnp.int32),
            pltpu.VMEM((D, BPW), jnp.float32),
            pltpu.VMEM((D, BPW), jnp.float32),
            pltpu.VMEM((D, BPW), jnp.float32),
            pltpu.VMEM((BPW,), jnp.float32),
            pltpu.VMEM((BPW,), jnp.float32),
            pltpu.VMEM((BPW,), jnp.float32),
            pltpu.VMEM((BPW,), jnp.float32),
            pltpu.VMEM((BPW,), jnp.float32),
            pltpu.SemaphoreType.DMA,
        ],
    )
    pos_preds, neg_preds = f(users3, pos3, neg3, uet, iet, ub1, ib1)
    return pos_preds, neg_preds


# speedup vs baseline: 5.5699x; 5.5699x over previous
import jax
import jax.numpy as jnp
from jax import lax
from jax.experimental import pallas as pl
from jax.experimental.pallas import tpu as pltpu
from jax.experimental.pallas import tpu_sc as plsc

NC = 2
NS = 16
L = 16
NW = NC * NS
B = 16384
D = 32
W = 128
BPW = B // NW
CHUNK = 128
NCH = BPW // CHUNK
GPC = CHUNK // L


def _sc_kernel(users_hbm, pos_hbm, neg_hbm, ue_hbm, ie_hbm, ub_hbm, ib_hbm,
               outp_hbm, outn_hbm,
               uidx, pidx, nidx, ue_rows, pe_rows, ne_rows,
               ub_v, pb_v, nb_v, outp_v, outn_v, sem, bsem):
    wid = lax.axis_index("core") * NS + lax.axis_index("subcore")

    pltpu.sync_copy(users_hbm.at[wid], uidx)
    pltpu.sync_copy(pos_hbm.at[wid], pidx)
    pltpu.sync_copy(neg_hbm.at[wid], nidx)

    bias_copies = []
    for j in range(NCH):
        sl = pl.ds(j * CHUNK, CHUNK)
        bias_copies.append(
            pltpu.async_copy(ub_hbm.at[uidx.at[j]], ub_v.at[sl], bsem))
        bias_copies.append(
            pltpu.async_copy(ib_hbm.at[pidx.at[j]], pb_v.at[sl], bsem))
        bias_copies.append(
            pltpu.async_copy(ib_hbm.at[nidx.at[j]], nb_v.at[sl], bsem))

    def fire(j, buf):
        return [
            pltpu.async_copy(ue_hbm.at[uidx.at[j]], ue_rows.at[buf], sem),
            pltpu.async_copy(ie_hbm.at[pidx.at[j]], pe_rows.at[buf], sem),
            pltpu.async_copy(ie_hbm.at[nidx.at[j]], ne_rows.at[buf], sem),
        ]

    lanes = lax.iota(jnp.int32, L)
    inflight = fire(0, 0)
    for j in range(NCH):
        for c in inflight:
            c.wait()
        if j + 1 < NCH:
            nxt = fire(j + 1, (j + 1) % 2)
        buf = j % 2

        for g in range(GPC):
            pos = g * L + lanes
            accp = jnp.zeros((L,), jnp.float32)
            accn = jnp.zeros((L,), jnp.float32)
            for dd in range(D):
                col = jnp.full((L,), dd, jnp.int32)
                u = plsc.load_gather(ue_rows.at[buf], [pos, col])
                p = plsc.load_gather(pe_rows.at[buf], [pos, col])
                n = plsc.load_gather(ne_rows.at[buf], [pos, col])
                accp = accp + u * p
                accn = accn + u * n
            sl = pl.ds(j * CHUNK + g * L, L)
            outp_v[sl] = accp
            outn_v[sl] = accn
        if j + 1 < NCH:
            inflight = nxt

    for c in bias_copies:
        c.wait()

    @pl.loop(0, BPW // L)
    def _bias(g):
        sl = pl.ds(g * L, L)
        ub = ub_v[sl]
        outp_v[sl] = outp_v[sl] + ub + pb_v[sl]
        outn_v[sl] = outn_v[sl] + ub + nb_v[sl]

    pltpu.sync_copy(outp_v, outp_hbm.at[pl.ds(wid * BPW, BPW)])
    pltpu.sync_copy(outn_v, outn_hbm.at[pl.ds(wid * BPW, BPW)])


def kernel(users, pos_items, neg_items, user_embeddings, item_embeddings,
           user_biases, item_biases):
    users3 = users.astype(jnp.int32).reshape(NW, NCH, CHUNK)
    pos3 = pos_items.astype(jnp.int32).reshape(NW, NCH, CHUNK)
    neg3 = neg_items.astype(jnp.int32).reshape(NW, NCH, CHUNK)
    uep = jnp.pad(user_embeddings, ((0, 0), (0, W - D)))
    iep = jnp.pad(item_embeddings, ((0, 0), (0, W - D)))
    ub1 = user_biases.reshape(-1)
    ib1 = item_biases.reshape(-1)

    mesh = plsc.VectorSubcoreMesh(core_axis_name="core",
                                  subcore_axis_name="subcore",
                                  num_cores=NC, num_subcores=NS)
    f = pl.kernel(
        _sc_kernel,
        compiler_params=pltpu.CompilerParams(needs_layout_passes=False,
                                             use_tc_tiling_on_sc=False),
        out_type=(jax.ShapeDtypeStruct((B,), jnp.float32),
                  jax.ShapeDtypeStruct((B,), jnp.float32)),
        mesh=mesh,
        scratch_types=[
            pltpu.VMEM((NCH, CHUNK), jnp.int32),
            pltpu.VMEM((NCH, CHUNK), jnp.int32),
            pltpu.VMEM((NCH, CHUNK), jnp.int32),
            pltpu.VMEM((2, CHUNK, W), jnp.float32),
            pltpu.VMEM((2, CHUNK, W), jnp.float32),
            pltpu.VMEM((2, CHUNK, W), jnp.float32),
            pltpu.VMEM((BPW,), jnp.float32),
            pltpu.VMEM((BPW,), jnp.float32),
            pltpu.VMEM((BPW,), jnp.float32),
            pltpu.VMEM((BPW,), jnp.float32),
            pltpu.VMEM((BPW,), jnp.float32),
            pltpu.SemaphoreType.DMA,
            pltpu.SemaphoreType.DMA,
        ],
    )
    pos_preds, neg_preds = f(users3, pos3, neg3, uep, iep, ub1, ib1)
    return pos_preds, neg_preds


# speedup vs baseline: 5.8182x; 1.0446x over previous
import jax
import jax.numpy as jnp
from jax import lax
from jax.experimental import pallas as pl
from jax.experimental.pallas import tpu as pltpu
from jax.experimental.pallas import tpu_sc as plsc

NC = 2
NS = 16
L = 16
NW = NC * NS
B = 16384
D = 32
BPW = B // NW
CHUNK = 128
NCH = BPW // CHUNK
GROUPS = BPW // L


def _sc_kernel(users_hbm, pos_hbm, neg_hbm, ue_hbm, ie_hbm, ub_hbm, ib_hbm,
               outp_hbm, outn_hbm,
               uidx, pidx, nidx, ue_rows, pe_rows, ne_rows,
               ub_v, pb_v, nb_v, outp_v, outn_v, sem):
    wid = lax.axis_index("core") * NS + lax.axis_index("subcore")

    pltpu.sync_copy(users_hbm.at[wid], uidx)
    pltpu.sync_copy(pos_hbm.at[wid], pidx)
    pltpu.sync_copy(neg_hbm.at[wid], nidx)

    copies = []
    for j in range(NCH):
        sl = pl.ds(j * CHUNK, CHUNK)
        copies.append(pltpu.async_copy(ue_hbm.at[uidx.at[j]], ue_rows.at[sl], sem))
        copies.append(pltpu.async_copy(ie_hbm.at[pidx.at[j]], pe_rows.at[sl], sem))
        copies.append(pltpu.async_copy(ie_hbm.at[nidx.at[j]], ne_rows.at[sl], sem))
        copies.append(pltpu.async_copy(ub_hbm.at[uidx.at[j]], ub_v.at[sl], sem))
        copies.append(pltpu.async_copy(ib_hbm.at[pidx.at[j]], pb_v.at[sl], sem))
        copies.append(pltpu.async_copy(ib_hbm.at[nidx.at[j]], nb_v.at[sl], sem))

    lanes = lax.iota(jnp.int32, L)
    GPC = CHUNK // L

    for j in range(NCH):
        for c in copies[6 * j:6 * (j + 1)]:
            c.wait()

        @pl.loop(0, GPC)
        def _group(g, j=j):
            base = j * CHUNK + g * L
            pos = base + lanes
            accp = jnp.zeros((L,), jnp.float32)
            accn = jnp.zeros((L,), jnp.float32)
            for dd in range(D):
                col = (lanes + dd) & (D - 1)
                u = plsc.load_gather(ue_rows, [pos, col])
                p = plsc.load_gather(pe_rows, [pos, col])
                n = plsc.load_gather(ne_rows, [pos, col])
                accp = accp + u * p
                accn = accn + u * n
            sl = pl.ds(base, L)
            ub = ub_v[sl]
            outp_v[sl] = accp + ub + pb_v[sl]
            outn_v[sl] = accn + ub + nb_v[sl]

    pltpu.sync_copy(outp_v, outp_hbm.at[pl.ds(wid * BPW, BPW)])
    pltpu.sync_copy(outn_v, outn_hbm.at[pl.ds(wid * BPW, BPW)])


def kernel(users, pos_items, neg_items, user_embeddings, item_embeddings,
           user_biases, item_biases):
    users3 = users.astype(jnp.int32).reshape(NW, NCH, CHUNK)
    pos3 = pos_items.astype(jnp.int32).reshape(NW, NCH, CHUNK)
    neg3 = neg_items.astype(jnp.int32).reshape(NW, NCH, CHUNK)
    ub1 = user_biases.reshape(-1)
    ib1 = item_biases.reshape(-1)

    mesh = plsc.VectorSubcoreMesh(core_axis_name="core",
                                  subcore_axis_name="subcore",
                                  num_cores=NC, num_subcores=NS)
    f = pl.kernel(
        _sc_kernel,
        compiler_params=pltpu.CompilerParams(needs_layout_passes=False,
                                             use_tc_tiling_on_sc=False),
        out_type=(jax.ShapeDtypeStruct((B,), jnp.float32),
                  jax.ShapeDtypeStruct((B,), jnp.float32)),
        mesh=mesh,
        scratch_types=[
            pltpu.VMEM((NCH, CHUNK), jnp.int32),
            pltpu.VMEM((NCH, CHUNK), jnp.int32),
            pltpu.VMEM((NCH, CHUNK), jnp.int32),
            pltpu.VMEM((BPW, D), jnp.float32),
            pltpu.VMEM((BPW, D), jnp.float32),
            pltpu.VMEM((BPW, D), jnp.float32),
            pltpu.VMEM((BPW,), jnp.float32),
            pltpu.VMEM((BPW,), jnp.float32),
            pltpu.VMEM((BPW,), jnp.float32),
            pltpu.VMEM((BPW,), jnp.float32),
            pltpu.VMEM((BPW,), jnp.float32),
            pltpu.SemaphoreType.DMA,
        ],
    )
    pos_preds, neg_preds = f(users3, pos3, neg3, user_embeddings,
                             item_embeddings, ub1, ib1)
    return pos_preds, neg_preds
